# baseline (speedup 1.0000x reference)
import functools

import jax
import jax.numpy as jnp
from jax import lax
from jax.experimental import pallas as pl
from jax.experimental.pallas import tpu as pltpu
from jax.experimental.pallas import tpu_sc as plsc

N_CLUSTERS = 100
EMBED_DIM = 64
BATCH = 16384

_D_SC = 16
_D_TC = EMBED_DIM - _D_SC
_NC = 2
_NS = 16
_NW = _NC * _NS
_B_PER_W = BATCH // _NW
_L = 16
_TC_BLK = 512


def _sc_embedding_gather(ids, table_flat_T):
    mesh = plsc.VectorSubcoreMesh(core_axis_name="c", subcore_axis_name="s")

    @functools.partial(
        pl.kernel,
        mesh=mesh,
        out_type=jax.ShapeDtypeStruct((_D_SC, BATCH), jnp.float32),
        scratch_types=[
            pltpu.VMEM((_B_PER_W,), jnp.int32),
            pltpu.VMEM((N_CLUSTERS * _D_SC,), jnp.float32),
            pltpu.VMEM((_D_SC, _B_PER_W), jnp.float32),
            pltpu.SemaphoreType.DMA,
        ],
        compiler_params=pltpu.CompilerParams(
            needs_layout_passes=False,
            skip_device_barrier=True,
            disable_bounds_checks=True,
            disable_semaphore_checks=True,
        ),
    )
    def k(ids_hbm, table_hbm, out_hbm, idx_v, tab_v, outT_v, sem_w):
        wid = lax.axis_index("s") * _NC + lax.axis_index("c")
        base = wid * _B_PER_W
        tab_cp = pltpu.async_copy(table_hbm, tab_v, sem_w)
        ids_cp = pltpu.async_copy(ids_hbm.at[pl.ds(base, _B_PER_W)], idx_v, sem_w)
        tab_cp.wait()
        ids_cp.wait()

        @plsc.parallel_loop(0, (_B_PER_W // _L) * (_D_SC // 8), unroll=2)
        def _body(u):
            bg = u >> 1
            dg = u & 1
            col = bg * _L
            bvec = idx_v[pl.ds(col, _L)]
            dvec = bvec + dg * (8 * N_CLUSTERS)
            for kk in range(8):
                ivec = dvec + kk * N_CLUSTERS
                outT_v[dg * 8 + kk, pl.ds(col, _L)] = plsc.load_gather(
                    tab_v, [ivec]
                )

        pltpu.async_copy(
            outT_v,
            out_hbm.at[:, pl.ds(base, _B_PER_W)],
            sem_w,
        ).wait()

    return k(ids, table_flat_T)


def _tc_onehot_body(ids_ref, tabT_ref, out_ref):
    ids_blk = ids_ref[...].reshape(1, _TC_BLK)
    rows = lax.broadcasted_iota(jnp.int32, (N_CLUSTERS, _TC_BLK), 0)
    onehot = jnp.where(rows == ids_blk, 1.0, 0.0).astype(jnp.float32)
    lhs = tabT_ref[pl.ds(_D_SC, _D_TC), :]
    out_ref[...] = jnp.dot(lhs, onehot, preferred_element_type=jnp.float32)


def _tc_onehot_matmul(ids, tableT):
    return pl.pallas_call(
        _tc_onehot_body,
        grid=(BATCH // _TC_BLK,),
        in_specs=[
            pl.BlockSpec((_TC_BLK,), lambda j: (j,)),
            pl.BlockSpec((EMBED_DIM, N_CLUSTERS), lambda j: (0, 0)),
        ],
        out_specs=pl.BlockSpec((_D_TC, _TC_BLK), lambda j: (0, j)),
        out_shape=jax.ShapeDtypeStruct((_D_TC, BATCH), jnp.float32),
    )(ids, tableT)


def kernel(cluster_ids, embedding_weight):
    ids = cluster_ids.astype(jnp.int32)
    tableT = embedding_weight.T
    table_flat_sc = tableT[:_D_SC].reshape(-1)
    outT_sc = _sc_embedding_gather(ids, table_flat_sc)
    outT_tc = _tc_onehot_matmul(ids, tableT)
    outT = jnp.concatenate([outT_sc, outT_tc], axis=0)
    return outT.T


# speedup vs baseline: 1.5595x; 1.5595x over previous
import functools

import jax
import jax.numpy as jnp
from jax import lax
from jax.experimental import pallas as pl
from jax.experimental.pallas import tpu as pltpu
from jax.experimental.pallas import tpu_sc as plsc

N_CLUSTERS = 100
EMBED_DIM = 64
BATCH = 16384

_NC = 2
_NS = 16
_NW = _NC * _NS
_B_PER_W = BATCH // _NW
_L = 16
_BLK = 128
_NBLK = _B_PER_W // _BLK


def _sc_embedding_gather(ids, table_flat_T):
    mesh = plsc.VectorSubcoreMesh(core_axis_name="c", subcore_axis_name="s")

    @functools.partial(
        pl.kernel,
        mesh=mesh,
        out_type=jax.ShapeDtypeStruct((EMBED_DIM, BATCH), jnp.float32),
        scratch_types=[
            pltpu.VMEM((_B_PER_W,), jnp.int32),
            pltpu.VMEM((N_CLUSTERS * EMBED_DIM,), jnp.float32),
            pltpu.VMEM((EMBED_DIM, _B_PER_W), jnp.float32),
            pltpu.SemaphoreType.DMA,
        ],
        compiler_params=pltpu.CompilerParams(
            needs_layout_passes=False,
            skip_device_barrier=True,
            disable_bounds_checks=True,
            disable_semaphore_checks=True,
        ),
    )
    def k(ids_hbm, table_hbm, out_hbm, idx_v, tab_v, outT_v, sem_w):
        wid = lax.axis_index("s") * _NC + lax.axis_index("c")
        base = wid * _B_PER_W
        tab_cp = pltpu.async_copy(table_hbm, tab_v, sem_w)
        ids_cp = pltpu.async_copy(ids_hbm.at[pl.ds(base, _B_PER_W)], idx_v, sem_w)
        tab_cp.wait()
        ids_cp.wait()

        @plsc.parallel_loop(0, (_B_PER_W // _L) * (EMBED_DIM // 8), unroll=4)
        def _body(u):
            bg = u >> 3
            dg = u & 7
            col = bg * _L
            bvec = idx_v[pl.ds(col, _L)]
            dvec = bvec + dg * (8 * N_CLUSTERS)
            for kk in range(8):
                ivec = dvec + kk * N_CLUSTERS
                outT_v[dg * 8 + kk, pl.ds(col, _L)] = plsc.load_gather(
                    tab_v, [ivec]
                )

        pltpu.async_copy(
            outT_v,
            out_hbm.at[:, pl.ds(base, _B_PER_W)],
            sem_w,
        ).wait()

    return k(ids, table_flat_T)


def kernel(cluster_ids, embedding_weight):
    ids = cluster_ids.astype(jnp.int32)
    table_flat_T = embedding_weight.T.reshape(-1)
    outT = _sc_embedding_gather(ids, table_flat_T)
    return outT.T


# speedup vs baseline: 1.5637x; 1.0027x over previous
import functools

import jax
import jax.numpy as jnp
from jax import lax
from jax.experimental import pallas as pl
from jax.experimental.pallas import tpu as pltpu
from jax.experimental.pallas import tpu_sc as plsc

N_CLUSTERS = 100
EMBED_DIM = 64
BATCH = 16384

_NC = 2
_NS = 16
_NW = _NC * _NS
_B_PER_W = BATCH // _NW
_L = 16
_BLK = 128
_NBLK = _B_PER_W // _BLK


def _sc_embedding_gather(ids, table_flat_T):
    mesh = plsc.VectorSubcoreMesh(core_axis_name="c", subcore_axis_name="s")

    @functools.partial(
        pl.kernel,
        mesh=mesh,
        out_type=jax.ShapeDtypeStruct((EMBED_DIM, BATCH), jnp.float32),
        scratch_types=[
            pltpu.VMEM((_B_PER_W,), jnp.int32),
            pltpu.VMEM((N_CLUSTERS * EMBED_DIM,), jnp.float32),
            pltpu.VMEM((EMBED_DIM, _B_PER_W), jnp.float32),
            pltpu.SemaphoreType.DMA,
        ],
        compiler_params=pltpu.CompilerParams(
            needs_layout_passes=False,
            skip_device_barrier=True,
            disable_bounds_checks=True,
            disable_semaphore_checks=True,
        ),
    )
    def k(ids_hbm, table_hbm, out_hbm, idx_v, tab_v, outT_v, sem_w):
        wid = lax.axis_index("s") * _NC + lax.axis_index("c")
        base = wid * _B_PER_W
        tab_cp = pltpu.async_copy(table_hbm, tab_v, sem_w)
        ids_cp = pltpu.async_copy(ids_hbm.at[pl.ds(base, _B_PER_W)], idx_v, sem_w)
        tab_cp.wait()
        ids_cp.wait()

        @plsc.parallel_loop(0, (_B_PER_W // _L) * (EMBED_DIM // 16), unroll=2)
        def _body(u):
            bg = u >> 2
            dg = u & 3
            col = bg * _L
            bvec = idx_v[pl.ds(col, _L)]
            dvec = bvec + dg * (16 * N_CLUSTERS)
            for kk in range(16):
                ivec = dvec + kk * N_CLUSTERS
                outT_v[dg * 16 + kk, pl.ds(col, _L)] = plsc.load_gather(
                    tab_v, [ivec]
                )

        pltpu.async_copy(
            outT_v,
            out_hbm.at[:, pl.ds(base, _B_PER_W)],
            sem_w,
        ).wait()

    return k(ids, table_flat_T)


def kernel(cluster_ids, embedding_weight):
    ids = cluster_ids.astype(jnp.int32)
    table_flat_T = embedding_weight.T.reshape(-1)
    outT = _sc_embedding_gather(ids, table_flat_T)
    return outT.T
